# baseline (speedup 1.0000x reference)
import jax
import jax.numpy as jnp
from jax import lax
from jax.experimental import pallas as pl
from jax.experimental.pallas import tpu as pltpu

_EPS = 1e-5
_VMEM_LIMIT = 64 * 1024 * 1024
_R_STATS = 4
_R_APPLY = 2


def _mm(a, b):
    return lax.dot_general(a, b, (((1,), (0,)), ((), ())),
                           preferred_element_type=jnp.float32)


def _row_phases(x, w_ref):
    cin, l = x.shape
    zcol = jnp.zeros((cin, 1), jnp.bfloat16)
    x_prev = jnp.concatenate([zcol, x[:, :l - 1]], axis=1)
    x_next = jnp.concatenate([x[:, 1:], zcol], axis=1)
    even = _mm(w_ref[0], x_prev) + _mm(w_ref[1], x)
    odd = _mm(w_ref[2], x) + _mm(w_ref[3], x_next)
    return even, odd


def _stats_kernel(x_ref, w_ref, stats_ref, xb_ref):
    cin, l = x_ref.shape[1], x_ref.shape[2]
    ones = jnp.ones((l, 1), jnp.float32)
    acc = None
    for r in range(x_ref.shape[0]):
        xr = x_ref[r].astype(jnp.bfloat16)
        xb_ref[r] = xr
        even, odd = _row_phases(xr, w_ref)
        xs = _mm(x_ref[r], ones)
        xsp = (xs - x_ref[r, :, l - 1:l]).astype(jnp.bfloat16)
        xsn = (xs - x_ref[r, :, 0:1]).astype(jnp.bfloat16)
        xsb = xs.astype(jnp.bfloat16)
        s1 = (_mm(w_ref[0], xsp) + _mm(w_ref[1], xsb)
              + _mm(w_ref[2], xsb) + _mm(w_ref[3], xsn))
        s2 = _mm(even * even, ones) + _mm(odd * odd, ones)
        s12 = jnp.concatenate([s1, s2], axis=1)
        acc = s12 if acc is None else acc + s12
    stats_ref[0] = acc


def _apply_kernel(xb_ref, w_ref, stats_ref, gamma_ref, beta_ref, cnt_ref,
                  d_ref, out_ref):
    s = jnp.sum(stats_ref[...], axis=0)
    cnt = cnt_ref[0, 0]
    mean = s[:, 0:1] / cnt
    var = jnp.maximum(s[:, 1:2] / cnt - mean * mean, 0.0)
    sc = gamma_ref[...] * lax.rsqrt(var + _EPS)
    sh = beta_ref[...] - mean * sc
    d = d_ref[...]
    for r in range(xb_ref.shape[0]):
        even, odd = _row_phases(xb_ref[r], w_ref)
        even = jnp.maximum(even * sc + sh, 0.0).astype(jnp.bfloat16)
        odd = jnp.maximum(odd * sc + sh, 0.0).astype(jnp.bfloat16)
        l = even.shape[1]
        for m in range(l // 128):
            pair = jnp.concatenate(
                [even[:, m * 128:(m + 1) * 128],
                 odd[:, m * 128:(m + 1) * 128]], axis=1)
            out_ref[r, :, m * 256:(m + 1) * 256] = _mm(pair, d)


def kernel(x, w, b, gamma, beta):
    del b
    x = x.astype(jnp.float32)
    n, cin, l = x.shape
    cout = w.shape[0]

    wf = w.astype(jnp.float32)
    w0, w1, w2 = wf[:, :, 0], wf[:, :, 1], wf[:, :, 2]
    w_pack = jnp.stack([w0, w1 + w2, w0 + w1, w2],
                       axis=0).astype(jnp.bfloat16)

    rs = _R_STATS if n % (2 * _R_STATS) == 0 else 1
    bs = n // (2 * rs)
    stats, xb = pl.pallas_call(
        _stats_kernel,
        grid=(2, bs),
        in_specs=[pl.BlockSpec((rs, cin, l), lambda c, i: (c * bs + i, 0, 0)),
                  pl.BlockSpec((4, cout, cin), lambda c, i: (0, 0, 0))],
        out_specs=[pl.BlockSpec((1, cout, 2), lambda c, i: (c * bs + i, 0, 0)),
                   pl.BlockSpec((rs, cin, l), lambda c, i: (c * bs + i, 0, 0))],
        out_shape=(jax.ShapeDtypeStruct((n // rs, cout, 2), jnp.float32),
                   jax.ShapeDtypeStruct((n, cin, l), jnp.bfloat16)),
        compiler_params=pltpu.CompilerParams(
            dimension_semantics=("parallel", "arbitrary"),
            vmem_limit_bytes=_VMEM_LIMIT),
    )(x, w_pack)

    r = jnp.arange(256)
    col = jnp.where(r < 128, 2 * r, 2 * (r - 128) + 1)
    d_perm = (col[:, None] == r[None, :]).astype(jnp.bfloat16)
    cnt = jnp.full((1, 1), float(n * 2 * l), jnp.float32)
    nb = n // rs

    ra = _R_APPLY if n % (2 * _R_APPLY) == 0 else 1
    ba = n // (2 * ra)
    out = pl.pallas_call(
        _apply_kernel,
        grid=(2, ba),
        in_specs=[pl.BlockSpec((ra, cin, l), lambda c, i: (c * ba + i, 0, 0)),
                  pl.BlockSpec((4, cout, cin), lambda c, i: (0, 0, 0)),
                  pl.BlockSpec((nb, cout, 2), lambda c, i: (0, 0, 0)),
                  pl.BlockSpec((cout, 1), lambda c, i: (0, 0)),
                  pl.BlockSpec((cout, 1), lambda c, i: (0, 0)),
                  pl.BlockSpec((1, 1), lambda c, i: (0, 0)),
                  pl.BlockSpec((256, 256), lambda c, i: (0, 0))],
        out_specs=pl.BlockSpec((ra, cout, 2 * l),
                               lambda c, i: (c * ba + i, 0, 0)),
        out_shape=jax.ShapeDtypeStruct((n, cout, 2 * l), jnp.float32),
        compiler_params=pltpu.CompilerParams(
            dimension_semantics=("parallel", "arbitrary"),
            vmem_limit_bytes=_VMEM_LIMIT),
    )(xb, w_pack, stats, gamma.astype(jnp.float32).reshape(cout, 1),
      beta.astype(jnp.float32).reshape(cout, 1), cnt, d_perm)
    return out


# speedup vs baseline: 1.2592x; 1.2592x over previous
import jax
import jax.numpy as jnp
from jax import lax
from jax.experimental import pallas as pl
from jax.experimental.pallas import tpu as pltpu

_EPS = 1e-5
_VMEM_LIMIT = 64 * 1024 * 1024
_R_STATS = 8
_R_APPLY = 2


def _mm(a, b):
    return lax.dot_general(a, b, (((1,), (0,)), ((), ())),
                           preferred_element_type=jnp.float32)


def _row_phases(x, w_ref):
    cin, l = x.shape
    zcol = jnp.zeros((cin, 1), jnp.bfloat16)
    x_prev = jnp.concatenate([zcol, x[:, :l - 1]], axis=1)
    x_next = jnp.concatenate([x[:, 1:], zcol], axis=1)
    even = _mm(w_ref[0], x_prev) + _mm(w_ref[1], x)
    odd = _mm(w_ref[2], x) + _mm(w_ref[3], x_next)
    return even, odd


def _stats_kernel(x_ref, w_ref, stats_ref, xb_ref):
    acc = None
    for r in range(x_ref.shape[0]):
        xr = x_ref[r].astype(jnp.bfloat16)
        xb_ref[r] = xr
        even, odd = _row_phases(xr, w_ref)
        s1 = (jnp.sum(even, axis=1, keepdims=True)
              + jnp.sum(odd, axis=1, keepdims=True))
        s2 = (jnp.sum(even * even, axis=1, keepdims=True)
              + jnp.sum(odd * odd, axis=1, keepdims=True))
        s12 = jnp.concatenate([s1, s2], axis=1)
        acc = s12 if acc is None else acc + s12
    stats_ref[0] = acc


def _apply_kernel(xb_ref, w_ref, stats_ref, gamma_ref, beta_ref, cnt_ref,
                  d_ref, out_ref):
    s = jnp.sum(stats_ref[...], axis=0)
    cnt = cnt_ref[0, 0]
    mean = s[:, 0:1] / cnt
    var = jnp.maximum(s[:, 1:2] / cnt - mean * mean, 0.0)
    sc = gamma_ref[...] * lax.rsqrt(var + _EPS)
    sh = beta_ref[...] - mean * sc
    d = d_ref[...]
    for r in range(xb_ref.shape[0]):
        even, odd = _row_phases(xb_ref[r], w_ref)
        even = jnp.maximum(even * sc + sh, 0.0).astype(jnp.bfloat16)
        odd = jnp.maximum(odd * sc + sh, 0.0).astype(jnp.bfloat16)
        l = even.shape[1]
        for m in range(l // 128):
            pair = jnp.concatenate(
                [even[:, m * 128:(m + 1) * 128],
                 odd[:, m * 128:(m + 1) * 128]], axis=1)
            out_ref[r, :, m * 256:(m + 1) * 256] = _mm(pair, d)


def kernel(x, w, b, gamma, beta):
    del b
    x = x.astype(jnp.float32)
    n, cin, l = x.shape
    cout = w.shape[0]

    wf = w.astype(jnp.float32)
    w0, w1, w2 = wf[:, :, 0], wf[:, :, 1], wf[:, :, 2]
    w_pack = jnp.stack([w0, w1 + w2, w0 + w1, w2],
                       axis=0).astype(jnp.bfloat16)

    rs = _R_STATS if n % (2 * _R_STATS) == 0 else 1
    bs = n // (2 * rs)
    stats, xb = pl.pallas_call(
        _stats_kernel,
        grid=(2, bs),
        in_specs=[pl.BlockSpec((rs, cin, l), lambda c, i: (c * bs + i, 0, 0)),
                  pl.BlockSpec((4, cout, cin), lambda c, i: (0, 0, 0))],
        out_specs=[pl.BlockSpec((1, cout, 2), lambda c, i: (c * bs + i, 0, 0)),
                   pl.BlockSpec((rs, cin, l), lambda c, i: (c * bs + i, 0, 0))],
        out_shape=(jax.ShapeDtypeStruct((n // rs, cout, 2), jnp.float32),
                   jax.ShapeDtypeStruct((n, cin, l), jnp.bfloat16)),
        compiler_params=pltpu.CompilerParams(
            dimension_semantics=("parallel", "arbitrary"),
            vmem_limit_bytes=_VMEM_LIMIT),
    )(x, w_pack)

    r = jnp.arange(256)
    col = jnp.where(r < 128, 2 * r, 2 * (r - 128) + 1)
    d_perm = (col[:, None] == r[None, :]).astype(jnp.bfloat16)
    cnt = jnp.full((1, 1), float(n * 2 * l), jnp.float32)
    nb = n // rs

    ra = _R_APPLY if n % (2 * _R_APPLY) == 0 else 1
    ba = n // (2 * ra)
    out = pl.pallas_call(
        _apply_kernel,
        grid=(2, ba),
        in_specs=[pl.BlockSpec((ra, cin, l), lambda c, i: (c * ba + i, 0, 0)),
                  pl.BlockSpec((4, cout, cin), lambda c, i: (0, 0, 0)),
                  pl.BlockSpec((nb, cout, 2), lambda c, i: (0, 0, 0)),
                  pl.BlockSpec((cout, 1), lambda c, i: (0, 0)),
                  pl.BlockSpec((cout, 1), lambda c, i: (0, 0)),
                  pl.BlockSpec((1, 1), lambda c, i: (0, 0)),
                  pl.BlockSpec((256, 256), lambda c, i: (0, 0))],
        out_specs=pl.BlockSpec((ra, cout, 2 * l),
                               lambda c, i: (c * ba + i, 0, 0)),
        out_shape=jax.ShapeDtypeStruct((n, cout, 2 * l), jnp.float32),
        compiler_params=pltpu.CompilerParams(
            dimension_semantics=("parallel", "arbitrary"),
            vmem_limit_bytes=_VMEM_LIMIT),
    )(xb, w_pack, stats, gamma.astype(jnp.float32).reshape(cout, 1),
      beta.astype(jnp.float32).reshape(cout, 1), cnt, d_perm)
    return out


# speedup vs baseline: 1.2728x; 1.0108x over previous
import jax
import jax.numpy as jnp
from jax import lax
from jax.experimental import pallas as pl
from jax.experimental.pallas import tpu as pltpu

_EPS = 1e-5
_VMEM_LIMIT = 64 * 1024 * 1024
_R_STATS = 4
_R_APPLY = 2


def _mm(a, b):
    return lax.dot_general(a, b, (((1,), (0,)), ((), ())),
                           preferred_element_type=jnp.float32)


def _row_phases(x, w_ref):
    cin, l = x.shape
    zcol = jnp.zeros((cin, 1), jnp.bfloat16)
    x_prev = jnp.concatenate([zcol, x[:, :l - 1]], axis=1)
    x_next = jnp.concatenate([x[:, 1:], zcol], axis=1)
    even = _mm(w_ref[0], x_prev) + _mm(w_ref[1], x)
    odd = _mm(w_ref[2], x) + _mm(w_ref[3], x_next)
    return even, odd


def _stats_kernel(x_ref, w_ref, stats_ref, xb_ref):
    acc = None
    for r in range(x_ref.shape[0]):
        xr = x_ref[r].astype(jnp.bfloat16)
        xb_ref[r] = xr
        even, odd = _row_phases(xr, w_ref)
        s1 = (jnp.sum(even, axis=1, keepdims=True)
              + jnp.sum(odd, axis=1, keepdims=True))
        s2 = (jnp.sum(even * even, axis=1, keepdims=True)
              + jnp.sum(odd * odd, axis=1, keepdims=True))
        s12 = jnp.concatenate([s1, s2], axis=1)
        acc = s12 if acc is None else acc + s12
    stats_ref[0] = acc


def _apply_kernel(xb_ref, w_ref, stats_ref, gamma_ref, beta_ref, cnt_ref,
                  d_ref, out_ref):
    s = jnp.sum(stats_ref[...], axis=0)
    cnt = cnt_ref[0, 0]
    mean = s[:, 0:1] / cnt
    var = jnp.maximum(s[:, 1:2] / cnt - mean * mean, 0.0)
    sc = gamma_ref[...] * lax.rsqrt(var + _EPS)
    sh = beta_ref[...] - mean * sc
    d = d_ref[...]
    for r in range(xb_ref.shape[0]):
        even, odd = _row_phases(xb_ref[r], w_ref)
        even = jnp.maximum(even * sc + sh, 0.0).astype(jnp.bfloat16)
        odd = jnp.maximum(odd * sc + sh, 0.0).astype(jnp.bfloat16)
        l = even.shape[1]
        for m in range(l // 128):
            pair = jnp.concatenate(
                [even[:, m * 128:(m + 1) * 128],
                 odd[:, m * 128:(m + 1) * 128]], axis=1)
            out_ref[r, :, m * 256:(m + 1) * 256] = _mm(pair, d)


def kernel(x, w, b, gamma, beta):
    del b
    x = x.astype(jnp.float32)
    n, cin, l = x.shape
    cout = w.shape[0]

    wf = w.astype(jnp.float32)
    w0, w1, w2 = wf[:, :, 0], wf[:, :, 1], wf[:, :, 2]
    w_pack = jnp.stack([w0, w1 + w2, w0 + w1, w2],
                       axis=0).astype(jnp.bfloat16)

    rs = _R_STATS if n % (2 * _R_STATS) == 0 else 1
    bs = n // (2 * rs)
    stats, xb = pl.pallas_call(
        _stats_kernel,
        grid=(2, bs),
        in_specs=[pl.BlockSpec((rs, cin, l), lambda c, i: (c * bs + i, 0, 0)),
                  pl.BlockSpec((4, cout, cin), lambda c, i: (0, 0, 0))],
        out_specs=[pl.BlockSpec((1, cout, 2), lambda c, i: (c * bs + i, 0, 0)),
                   pl.BlockSpec((rs, cin, l), lambda c, i: (c * bs + i, 0, 0))],
        out_shape=(jax.ShapeDtypeStruct((n // rs, cout, 2), jnp.float32),
                   jax.ShapeDtypeStruct((n, cin, l), jnp.bfloat16)),
        compiler_params=pltpu.CompilerParams(
            dimension_semantics=("parallel", "arbitrary"),
            vmem_limit_bytes=_VMEM_LIMIT),
    )(x, w_pack)

    r = jnp.arange(256)
    col = jnp.where(r < 128, 2 * r, 2 * (r - 128) + 1)
    d_perm = (col[:, None] == r[None, :]).astype(jnp.bfloat16)
    cnt = jnp.full((1, 1), float(n * 2 * l), jnp.float32)
    nb = n // rs

    ra = _R_APPLY if n % (2 * _R_APPLY) == 0 else 1
    ba = n // (2 * ra)
    out = pl.pallas_call(
        _apply_kernel,
        grid=(2, ba),
        in_specs=[pl.BlockSpec((ra, cin, l), lambda c, i: (c * ba + i, 0, 0)),
                  pl.BlockSpec((4, cout, cin), lambda c, i: (0, 0, 0)),
                  pl.BlockSpec((nb, cout, 2), lambda c, i: (0, 0, 0)),
                  pl.BlockSpec((cout, 1), lambda c, i: (0, 0)),
                  pl.BlockSpec((cout, 1), lambda c, i: (0, 0)),
                  pl.BlockSpec((1, 1), lambda c, i: (0, 0)),
                  pl.BlockSpec((256, 256), lambda c, i: (0, 0))],
        out_specs=pl.BlockSpec((ra, cout, 2 * l),
                               lambda c, i: (c * ba + i, 0, 0)),
        out_shape=jax.ShapeDtypeStruct((n, cout, 2 * l), jnp.float32),
        compiler_params=pltpu.CompilerParams(
            dimension_semantics=("parallel", "arbitrary"),
            vmem_limit_bytes=_VMEM_LIMIT),
    )(xb, w_pack, stats, gamma.astype(jnp.float32).reshape(cout, 1),
      beta.astype(jnp.float32).reshape(cout, 1), cnt, d_perm)
    return out
